# baseline (speedup 1.0000x reference)
import jax
import jax.numpy as jnp
from jax.experimental import pallas as pl

_TAU = 0.3
_BB = 8


def _body(logits_ref, u_ref, out_ref):
    k = pl.program_id(1)
    u = u_ref[...]
    lg = logits_ref[...]
    g = -jnp.log(-jnp.log(u))
    x = (g + lg) / _TAU
    m = jnp.max(x, axis=1, keepdims=True)
    e = jnp.exp(x - m)
    s = jnp.sum(e, axis=1, keepdims=True)
    p = e / s

    @pl.when(k == 0)
    def _():
        out_ref[...] = p

    @pl.when(k > 0)
    def _():
        out_ref[...] = jnp.maximum(out_ref[...], p)


def kernel(logits, uniform):
    B, D = logits.shape
    K = uniform.shape[1]
    u2 = uniform.reshape(B, K * D)
    return pl.pallas_call(
        _body,
        grid=(B // _BB, K),
        in_specs=[
            pl.BlockSpec((_BB, D), lambda b, k: (b, 0)),
            pl.BlockSpec((_BB, D), lambda b, k: (b, k)),
        ],
        out_specs=pl.BlockSpec((_BB, D), lambda b, k: (b, 0)),
        out_shape=jax.ShapeDtypeStruct((B, D), jnp.float32),
    )(logits, u2)


# speedup vs baseline: 1.9460x; 1.9460x over previous
import jax
import jax.numpy as jnp
from jax.experimental import pallas as pl
from jax.experimental.pallas import tpu as pltpu

_TAU = 0.3
_BB = 8
_NSLOT = 6
_GRP = 4
_LOG2E = 1.4426950408889634


def _make_body(B, K, D):
    nb = B // _BB
    nsheets = nb * K

    def _body(lg_ref, u_hbm, out_ref, ubuf, sem):
        b = pl.program_id(0)

        def copy_for(i, slot):
            bb = i // K
            kk = i - bb * K
            return pltpu.make_async_copy(
                u_hbm.at[pl.ds(bb * _BB, _BB), kk, :], ubuf.at[slot],
                sem.at[slot])

        @pl.when(b == 0)
        def _():
            for j in range(_NSLOT):
                copy_for(j, j).start()

        lgt2 = lg_ref[...] * (_LOG2E / _TAU)

        def sheet(i, slot):
            u = ubuf[slot]
            w = -jnp.log(u)
            x2 = lgt2 - jnp.log(w) * (_LOG2E / _TAU)
            m2 = jnp.max(x2, axis=1, keepdims=True)
            e = jnp.exp2(x2 - m2)
            s = jnp.sum(e, axis=1, keepdims=True)
            return e / s

        acc = None
        for k in range(0, K, _GRP):
            idx = [b * K + k + j for j in range(_GRP)]
            for i in idx:
                copy_for(i, i % _NSLOT).wait()
            ps = [sheet(i, i % _NSLOT) for i in idx]
            while len(ps) > 1:
                ps = [jnp.maximum(a, c) for a, c in zip(ps[::2], ps[1::2])]
            acc = ps[0] if acc is None else jnp.maximum(acc, ps[0])

            for i in idx:
                @pl.when(i + _NSLOT < nsheets)
                def _(i=i + _NSLOT):
                    copy_for(i, i % _NSLOT).start()

        out_ref[...] = acc

    return _body


def kernel(logits, uniform):
    B, D = logits.shape
    K = uniform.shape[1]
    return pl.pallas_call(
        _make_body(B, K, D),
        grid=(B // _BB,),
        in_specs=[
            pl.BlockSpec((_BB, D), lambda b: (b, 0)),
            pl.BlockSpec(memory_space=pltpu.MemorySpace.HBM),
        ],
        out_specs=pl.BlockSpec((_BB, D), lambda b: (b, 0)),
        out_shape=jax.ShapeDtypeStruct((B, D), jnp.float32),
        scratch_shapes=[
            pltpu.VMEM((_NSLOT, _BB, D), jnp.float32),
            pltpu.SemaphoreType.DMA((_NSLOT,)),
        ],
    )(logits, uniform)


# speedup vs baseline: 2.3504x; 1.2078x over previous
import jax
import jax.numpy as jnp
from jax.experimental import pallas as pl
from jax.experimental.pallas import tpu as pltpu

_TAU = 0.3
_BB = 8
_LOG2E = 1.4426950408889634


def _body(logits_ref, u_ref, out_ref):
    lgt2 = logits_ref[...] * (_LOG2E / _TAU)
    u = u_ref[...]
    w = -jnp.log(u)
    x2 = lgt2[:, None, :] - jnp.log(w) * (_LOG2E / _TAU)
    m2 = jnp.max(x2, axis=2, keepdims=True)
    e = jnp.exp2(x2 - m2)
    s = jnp.sum(e, axis=2, keepdims=True)
    p = e / s
    out_ref[...] = jnp.max(p.transpose(1, 0, 2), axis=0)


def kernel(logits, uniform):
    B, D = logits.shape
    K = uniform.shape[1]
    return pl.pallas_call(
        _body,
        grid=(B // _BB,),
        in_specs=[
            pl.BlockSpec((_BB, D), lambda b: (b, 0)),
            pl.BlockSpec((_BB, K, D), lambda b: (b, 0, 0)),
        ],
        out_specs=pl.BlockSpec((_BB, D), lambda b: (b, 0)),
        out_shape=jax.ShapeDtypeStruct((B, D), jnp.float32),
        compiler_params=pltpu.CompilerParams(
            dimension_semantics=("parallel",)),
    )(logits, uniform)
